# baseline (speedup 1.0000x reference)
import functools
import jax
import jax.numpy as jnp
from jax import lax
from jax.experimental import pallas as pl
from jax.experimental.pallas import tpu as pltpu
from jax.experimental.pallas import tpu_sc as plsc

_B = 128
_NC = 2
_NS = 16
_NW = _NC * _NS
_REP = 4
_GRP = 8


def _sc_body(n_per_w, table_hbm, out_hbm, rep_v, sem):
    wid = lax.axis_index("s") * _NC + lax.axis_index("c")
    base = wid * n_per_w
    for r in range(_REP):
        pltpu.sync_copy(table_hbm.at[pl.ds(base, n_per_w)], rep_v.at[r])

    nchunk = _B // _REP

    def group(g, carry):
        c0 = g * _GRP
        for j in range(_GRP):
            pltpu.make_async_copy(
                rep_v,
                out_hbm.at[pl.ds((c0 + j) * _REP, _REP), pl.ds(base, n_per_w)],
                sem,
            ).start()
        for j in range(_GRP):
            pltpu.make_async_copy(
                rep_v,
                out_hbm.at[pl.ds((c0 + j) * _REP, _REP), pl.ds(base, n_per_w)],
                sem,
            ).wait()
        return carry

    lax.fori_loop(0, nchunk // _GRP, group, 0)


def kernel(batch_size, table):
    n, d = table.shape
    n_per_w = n // _NW
    mesh = plsc.VectorSubcoreMesh(core_axis_name="c", subcore_axis_name="s")
    k = pl.kernel(
        functools.partial(_sc_body, n_per_w),
        out_type=jax.ShapeDtypeStruct((_B, n, d), table.dtype),
        mesh=mesh,
        scratch_types=[
            pltpu.VMEM((_REP, n_per_w, d), table.dtype),
            pltpu.SemaphoreType.DMA,
        ],
    )
    return k(table)
